# baseline (speedup 1.0000x reference)
import functools

import jax
import jax.numpy as jnp
import numpy as np
from jax import lax
from jax.experimental import pallas as pl
from jax.experimental.pallas import tpu as pltpu
from jax.experimental.pallas import tpu_sc as plsc

B = 16384
L = 50
USER_DIM = 128
CAT_DIM = 64
HIDDEN = 96
NUM_CATEGORIES = 1000

NC = 2
NS = 16
NW = NC * NS
BPW = B // NW

CH = 128
CAT_NCH = B * L // CH // NW
NBUF = 10
LOOKAHEAD = 5

_DEST_IDS = ((np.arange(B * L, dtype=np.int64) // L) % (B // NC)).astype(
    np.int32).reshape(B * L // CH, CH)

_vmesh = plsc.VectorSubcoreMesh(core_axis_name="c", subcore_axis_name="s")


def _user_gather(user_table, user_ids_2d):
  @functools.partial(
      pl.kernel,
      out_type=jax.ShapeDtypeStruct((B, USER_DIM), jnp.float32),
      mesh=_vmesh,
      scratch_types=[
          pltpu.VMEM((4, CH), jnp.int32),
          pltpu.VMEM((BPW, USER_DIM), jnp.float32),
      ],
  )
  def k(table_hbm, ids_hbm, out_hbm, idx_v, rows_v):
    wid = lax.axis_index("c") * NS + lax.axis_index("s")
    base = wid * BPW
    pltpu.sync_copy(ids_hbm.at[pl.ds(wid * 4, 4)], idx_v)
    for j in range(4):
      pltpu.sync_copy(table_hbm.at[idx_v.at[j]],
                      rows_v.at[pl.ds(j * CH, CH)])
    pltpu.sync_copy(rows_v, out_hbm.at[pl.ds(base, BPW)])

  return k(user_table, user_ids_2d)


def _cat_bag(category_table, cat_ids_2d, dest_ids_2d):
  @functools.partial(
      pl.kernel,
      out_type=jax.ShapeDtypeStruct((B, 2 * CAT_DIM), jnp.float32),
      mesh=_vmesh,
      compiler_params=pltpu.CompilerParams(use_tc_tiling_on_sc=False,
                                           needs_layout_passes=False),
      scratch_types=[
          pltpu.VMEM((CH, USER_DIM), jnp.float32),
          pltpu.VMEM((CAT_NCH, CH), jnp.int32),
          pltpu.VMEM((CAT_NCH, CH), jnp.int32),
          [pltpu.VMEM((CH, CAT_DIM), jnp.bfloat16) for _ in range(NBUF)],
          pltpu.VMEM_SHARED((B // NC, CAT_DIM), jnp.bfloat16),
          [pltpu.SemaphoreType.DMA for _ in range(NBUF)],
          [pltpu.SemaphoreType.DMA for _ in range(NBUF)],
      ],
  )
  def k(ctable_hbm, cids_hbm, dest_hbm, cout_hbm, urows_v, idx_v, dest_v,
        bufs, acc_sh, semg, sems):
    cid = lax.axis_index("c")
    sid = lax.axis_index("s")
    wid = cid * NS + sid
    base = wid * BPW

    pltpu.sync_copy(cids_hbm.at[pl.ds(wid * CAT_NCH, CAT_NCH)], idx_v)
    pltpu.sync_copy(dest_hbm.at[pl.ds(wid * CAT_NCH, CAT_NCH)], dest_v)

    zero32 = jnp.zeros((32,), jnp.bfloat16)

    @pl.loop(0, CH)
    def _(r):
      for g in range(CAT_DIM // 32):
        bufs[0][r, pl.ds(g * 32, 32)] = zero32

    for kk in range(BPW // CH):
      pltpu.sync_copy(bufs[0], acc_sh.at[pl.ds(sid * BPW + kk * CH, CH)])

    for c in range(LOOKAHEAD):
      pltpu.async_copy(ctable_hbm.at[idx_v.at[c]], bufs[c], semg[c])

    @pl.loop(0, CAT_NCH, step=NBUF)
    def _(j):
      for b in range(NBUF):
        cidx = j + b
        pltpu.make_async_copy(ctable_hbm.at[idx_v.at[0]], bufs[b],
                              semg[b]).wait()

        @pl.when(cidx >= LOOKAHEAD)
        def _():
          pltpu.make_async_copy(bufs[(b + LOOKAHEAD) % NBUF],
                                acc_sh.at[dest_v.at[0]],
                                sems[(b + LOOKAHEAD) % NBUF]).wait()

        @pl.when(cidx + LOOKAHEAD < CAT_NCH)
        def _():
          pltpu.async_copy(ctable_hbm.at[idx_v.at[cidx + LOOKAHEAD]],
                           bufs[(b + LOOKAHEAD) % NBUF],
                           semg[(b + LOOKAHEAD) % NBUF])

        pltpu.async_copy(bufs[b], acc_sh.at[dest_v.at[cidx]], sems[b],
                         add=True)

    for c in range(CAT_NCH - LOOKAHEAD, CAT_NCH):
      pltpu.make_async_copy(bufs[c % NBUF], acc_sh.at[dest_v.at[0]],
                            sems[c % NBUF]).wait()

    zero16f = jnp.zeros((16,), jnp.float32)

    @pl.loop(0, CH)
    def _(r):
      for g in range(CAT_DIM // 16, USER_DIM // 16):
        urows_v[r, pl.ds(g * 16, 16)] = zero16f

    iota2 = lax.iota(jnp.int32, 16) * 2
    for kk in range(BPW // CH):
      pltpu.sync_copy(acc_sh.at[pl.ds(sid * BPW + kk * CH, CH)], bufs[0])

      @pl.loop(0, CH)
      def _(r):
        rvec = jnp.full((16,), r, jnp.int32)
        for g in range(CAT_DIM // 32):
          v = plsc.bitcast(bufs[0][r, pl.ds(g * 32, 32)], jnp.int32)
          lo = plsc.bitcast(v << 16, jnp.float32)
          hi = plsc.bitcast(v & jnp.int32(-65536), jnp.float32)
          cols = iota2 + (g * 32)
          plsc.store_scatter(urows_v, [rvec, cols], lo)
          plsc.store_scatter(urows_v, [rvec, cols + 1], hi)

      pltpu.sync_copy(urows_v, cout_hbm.at[pl.ds(base + kk * CH, CH)])

  return k(category_table, cat_ids_2d, dest_ids_2d)


def _mlp(u, csum, W1u, W1c, b1, W2, b2):
  BB = 2048
  dot = functools.partial(jnp.dot, preferred_element_type=jnp.float32)

  def body(u_ref, c_ref, w1u_ref, w1c_ref, b1_ref, w2_ref, b2_ref, o_ref):
    ub = u_ref[...].astype(jnp.bfloat16)
    cb = c_ref[...].astype(jnp.bfloat16)
    x = dot(ub, w1u_ref[...]) + dot(cb, w1c_ref[...])
    x = jnp.maximum(x + b1_ref[...], 0.0).astype(jnp.bfloat16)
    z = dot(x, w2_ref[...]) + b2_ref[...]
    o_ref[...] = 0.5 * jnp.tanh(0.5 * z) + 0.5

  return pl.pallas_call(
      body,
      grid=(B // BB,),
      in_specs=[
          pl.BlockSpec((BB, USER_DIM), lambda i: (i, 0)),
          pl.BlockSpec((BB, 2 * CAT_DIM), lambda i: (i, 0)),
          pl.BlockSpec((USER_DIM, HIDDEN), lambda i: (0, 0)),
          pl.BlockSpec((2 * CAT_DIM, HIDDEN), lambda i: (0, 0)),
          pl.BlockSpec((1, HIDDEN), lambda i: (0, 0)),
          pl.BlockSpec((HIDDEN, NUM_CATEGORIES), lambda i: (0, 0)),
          pl.BlockSpec((1, NUM_CATEGORIES), lambda i: (0, 0)),
      ],
      out_specs=pl.BlockSpec((BB, NUM_CATEGORIES), lambda i: (i, 0)),
      out_shape=jax.ShapeDtypeStruct((B, NUM_CATEGORIES), jnp.float32),
  )(u, csum, W1u, W1c, b1.reshape(1, HIDDEN), W2,
    b2.reshape(1, NUM_CATEGORIES))


def kernel(user_ids, category_ids, user_table, category_table, W1, b1, W2, b2):
  u = _user_gather(user_table, user_ids.reshape(B // CH, CH))
  csum = _cat_bag(category_table.astype(jnp.bfloat16),
                  category_ids.reshape(B * L // CH, CH),
                  jnp.asarray(_DEST_IDS))
  W1u = W1[:USER_DIM].astype(jnp.bfloat16)
  W1c = (W1[USER_DIM:] * (1.0 / L)).astype(jnp.bfloat16)
  W1c = jnp.concatenate([W1c, jnp.zeros((CAT_DIM, HIDDEN), jnp.bfloat16)])
  return _mlp(u, csum, W1u, W1c, b1, W2.astype(jnp.bfloat16), b2)


# speedup vs baseline: 1.4064x; 1.4064x over previous
import functools

import jax
import jax.numpy as jnp
import numpy as np
from jax import lax
from jax.experimental import pallas as pl
from jax.experimental.pallas import tpu as pltpu
from jax.experimental.pallas import tpu_sc as plsc

B = 16384
L = 50
USER_DIM = 128
CAT_DIM = 64
HIDDEN = 96
NUM_CATEGORIES = 1000

NC = 2
NS = 16
NW = NC * NS
BPW = B // NW

CH = 128
CAT_NCH = B * L // CH // NW
NBUF = 10
LOOKAHEAD = 5

_DEST_IDS = ((np.arange(B * L, dtype=np.int64) // L) % (B // NC)).astype(
    np.int32).reshape(B * L // CH, CH)

_vmesh = plsc.VectorSubcoreMesh(core_axis_name="c", subcore_axis_name="s")


def _user_gather(user_table, user_ids_2d):
  @functools.partial(
      pl.kernel,
      out_type=jax.ShapeDtypeStruct((B, USER_DIM), jnp.float32),
      mesh=_vmesh,
      scratch_types=[
          pltpu.VMEM((4, CH), jnp.int32),
          pltpu.VMEM((BPW, USER_DIM), jnp.float32),
      ],
  )
  def k(table_hbm, ids_hbm, out_hbm, idx_v, rows_v):
    wid = lax.axis_index("c") * NS + lax.axis_index("s")
    base = wid * BPW
    pltpu.sync_copy(ids_hbm.at[pl.ds(wid * 4, 4)], idx_v)
    for j in range(4):
      pltpu.sync_copy(table_hbm.at[idx_v.at[j]],
                      rows_v.at[pl.ds(j * CH, CH)])
    pltpu.sync_copy(rows_v, out_hbm.at[pl.ds(base, BPW)])

  return k(user_table, user_ids_2d)


def _cat_bag(category_table, cat_ids_2d, dest_ids_2d):
  @functools.partial(
      pl.kernel,
      out_type=jax.ShapeDtypeStruct((B, 2 * CAT_DIM), jnp.float32),
      mesh=_vmesh,
      compiler_params=pltpu.CompilerParams(use_tc_tiling_on_sc=False,
                                           needs_layout_passes=False),
      scratch_types=[
          pltpu.VMEM((CH, USER_DIM), jnp.float32),
          pltpu.VMEM((CAT_NCH, CH), jnp.int32),
          pltpu.VMEM((CAT_NCH, CH), jnp.int32),
          [pltpu.VMEM((CH, CAT_DIM), jnp.bfloat16) for _ in range(NBUF)],
          pltpu.VMEM_SHARED((B // NC, CAT_DIM), jnp.bfloat16),
          [pltpu.SemaphoreType.DMA for _ in range(NBUF)],
          [pltpu.SemaphoreType.DMA for _ in range(NBUF)],
      ],
  )
  def k(ctable_hbm, cids_hbm, dest_hbm, cout_hbm, urows_v, idx_v, dest_v,
        bufs, acc_sh, semg, sems):
    cid = lax.axis_index("c")
    sid = lax.axis_index("s")
    wid = cid * NS + sid
    base = wid * BPW

    pltpu.sync_copy(cids_hbm.at[pl.ds(wid * CAT_NCH, CAT_NCH)], idx_v)
    pltpu.sync_copy(dest_hbm.at[pl.ds(wid * CAT_NCH, CAT_NCH)], dest_v)

    zero32 = jnp.zeros((32,), jnp.bfloat16)

    @pl.loop(0, CH)
    def _(r):
      for g in range(CAT_DIM // 32):
        bufs[0][r, pl.ds(g * 32, 32)] = zero32

    for kk in range(BPW // CH):
      pltpu.sync_copy(bufs[0], acc_sh.at[pl.ds(sid * BPW + kk * CH, CH)])

    for c in range(LOOKAHEAD):
      pltpu.async_copy(ctable_hbm.at[idx_v.at[c]], bufs[c], semg[c])

    @pl.loop(0, CAT_NCH, step=NBUF)
    def _(j):
      for b in range(NBUF):
        cidx = j + b
        pltpu.make_async_copy(ctable_hbm.at[idx_v.at[0]], bufs[b],
                              semg[b]).wait()

        @pl.when(cidx >= LOOKAHEAD)
        def _():
          pltpu.make_async_copy(bufs[(b + LOOKAHEAD) % NBUF],
                                acc_sh.at[dest_v.at[0]],
                                sems[(b + LOOKAHEAD) % NBUF]).wait()

        @pl.when(cidx + LOOKAHEAD < CAT_NCH)
        def _():
          pltpu.async_copy(ctable_hbm.at[idx_v.at[cidx + LOOKAHEAD]],
                           bufs[(b + LOOKAHEAD) % NBUF],
                           semg[(b + LOOKAHEAD) % NBUF])

        pltpu.async_copy(bufs[b], acc_sh.at[dest_v.at[cidx]], sems[b],
                         add=True)

    for c in range(CAT_NCH - LOOKAHEAD, CAT_NCH):
      pltpu.make_async_copy(bufs[c % NBUF], acc_sh.at[dest_v.at[0]],
                            sems[c % NBUF]).wait()

    zero16f = jnp.zeros((16,), jnp.float32)

    @pl.loop(0, CH)
    def _(r):
      for g in range(CAT_DIM // 16, USER_DIM // 16):
        urows_v[r, pl.ds(g * 16, 16)] = zero16f

    iota2 = lax.iota(jnp.int32, 16) * 2
    for kk in range(BPW // CH):
      pltpu.sync_copy(acc_sh.at[pl.ds(sid * BPW + kk * CH, CH)], bufs[0])

      @pl.loop(0, CH)
      def _(r):
        rvec = jnp.full((16,), r, jnp.int32)
        for g in range(CAT_DIM // 32):
          v = plsc.bitcast(bufs[0][r, pl.ds(g * 32, 32)], jnp.int32)
          lo = plsc.bitcast(v << 16, jnp.float32)
          hi = plsc.bitcast(v & jnp.int32(-65536), jnp.float32)
          cols = iota2 + (g * 32)
          plsc.store_scatter(urows_v, [rvec, cols], lo)
          plsc.store_scatter(urows_v, [rvec, cols + 1], hi)

      pltpu.sync_copy(urows_v, cout_hbm.at[pl.ds(base + kk * CH, CH)])

  return k(category_table, cat_ids_2d, dest_ids_2d)


def _mlp(u, csum, W1u_t, W1c_t, b1, W2_t, b2):
  BB = 2048

  def body(u_ref, c_ref, w1u_ref, w1c_ref, b1_ref, w2_ref, b2_ref, o_ref):
    ub = u_ref[...].astype(jnp.bfloat16)
    cb = c_ref[...].astype(jnp.bfloat16)
    dnr = (((1,), (1,)), ((), ()))
    dns = (((1,), (0,)), ((), ()))
    x_t = lax.dot_general(w1u_ref[...], ub, dnr,
                          preferred_element_type=jnp.float32)
    x_t = x_t + lax.dot_general(w1c_ref[...], cb, dnr,
                                preferred_element_type=jnp.float32)
    x_t = jnp.maximum(x_t + b1_ref[...], 0.0).astype(jnp.bfloat16)
    z_t = lax.dot_general(w2_ref[...], x_t, dns,
                          preferred_element_type=jnp.float32)
    o_ref[...] = 0.5 * jnp.tanh((z_t + b2_ref[...]) * 0.5) + 0.5

  return pl.pallas_call(
      body,
      grid=(B // BB,),
      in_specs=[
          pl.BlockSpec((BB, USER_DIM), lambda i: (i, 0)),
          pl.BlockSpec((BB, 2 * CAT_DIM), lambda i: (i, 0)),
          pl.BlockSpec((HIDDEN, USER_DIM), lambda i: (0, 0)),
          pl.BlockSpec((HIDDEN, 2 * CAT_DIM), lambda i: (0, 0)),
          pl.BlockSpec((HIDDEN, 1), lambda i: (0, 0)),
          pl.BlockSpec((NUM_CATEGORIES, HIDDEN), lambda i: (0, 0)),
          pl.BlockSpec((NUM_CATEGORIES, 1), lambda i: (0, 0)),
      ],
      out_specs=pl.BlockSpec((NUM_CATEGORIES, BB), lambda i: (0, i)),
      out_shape=jax.ShapeDtypeStruct((NUM_CATEGORIES, B), jnp.float32),
  )(u, csum, W1u_t, W1c_t, b1.reshape(HIDDEN, 1), W2_t,
    b2.reshape(NUM_CATEGORIES, 1))


def kernel(user_ids, category_ids, user_table, category_table, W1, b1, W2, b2):
  u = _user_gather(user_table, user_ids.reshape(B // CH, CH))
  csum = _cat_bag(category_table.astype(jnp.bfloat16),
                  category_ids.reshape(B * L // CH, CH),
                  jnp.asarray(_DEST_IDS))
  W1u_t = W1[:USER_DIM].T.astype(jnp.bfloat16)
  W1c_t = (W1[USER_DIM:] * (1.0 / L)).T.astype(jnp.bfloat16)
  W1c_t = jnp.concatenate(
      [W1c_t, jnp.zeros((HIDDEN, CAT_DIM), jnp.bfloat16)], axis=1)
  out_t = _mlp(u, csum, W1u_t, W1c_t, b1, W2.T.astype(jnp.bfloat16), b2)
  return out_t.T
